# Initial kernel scaffold; baseline (speedup 1.0000x reference)
import jax
import jax.numpy as jnp
from jax.experimental import pallas as pl


def kernel(node_feat, pos, edge_index, Wp, bp, W1e, b1e, W2e, b2e, Wc0, bc0, Wc1, bc1, Wc2, bc2):
    raise NotImplementedError("write your pallas kernel here")


# speedup vs baseline: 3.8234x; 3.8234x over previous
import functools

import jax
import jax.numpy as jnp
from jax import lax
from jax.experimental import pallas as pl
from jax.experimental.pallas import tpu as pltpu
from jax.experimental.pallas import tpu_sc as plsc

N = 10000
E = 320000
D = 128
NC = 2
NS = 16
NW = NC * NS
EPW = E // NW
K = 128
NB = EPW // K
REM = EPW - NB * K
NP = 10240
RPT = NP // NS
ZCH = 128

_mesh = plsc.VectorSubcoreMesh(core_axis_name="c", subcore_axis_name="s")
_sc_params = pltpu.CompilerParams(needs_layout_passes=False)


def _worker_base():
    c = lax.axis_index("c")
    s = lax.axis_index("s")
    return c, s, (c * NS + s) * EPW


def _fill_rows(ref, nrows, ncols, value):
    vec = jnp.full((16,), value, jnp.float32)

    @pl.loop(0, nrows)
    def _(r):
        for cc in range(ncols // 16):
            ref[r, pl.ds(cc * 16, 16)] = vec


def _zero_acc_slice(acc, zbuf, s):
    for k in range(RPT // ZCH):
        pltpu.sync_copy(zbuf, acc.at[pl.ds(s * RPT + k * ZCH, ZCH)])


def _read_out_acc(acc, zbuf, out_hbm, c, s):
    for k in range(RPT // ZCH):
        r0 = s * RPT + k * ZCH
        pltpu.sync_copy(acc.at[pl.ds(r0, ZCH)], zbuf)
        pltpu.sync_copy(zbuf, out_hbm.at[c, pl.ds(r0, ZCH)])


@functools.partial(
    pl.kernel,
    out_type=(
        jax.ShapeDtypeStruct((E,), jnp.float32),
        jax.ShapeDtypeStruct((E,), jnp.float32),
        jax.ShapeDtypeStruct((E,), jnp.float32),
    ),
    mesh=_mesh,
    compiler_params=_sc_params,
    scratch_types=(
        pltpu.VMEM((N,), jnp.float32),
        pltpu.VMEM((N,), jnp.float32),
        pltpu.VMEM((N,), jnp.float32),
        pltpu.VMEM((K,), jnp.int32),
        pltpu.VMEM((K,), jnp.int32),
        pltpu.VMEM((K,), jnp.float32),
        pltpu.VMEM((K,), jnp.float32),
        pltpu.VMEM((K,), jnp.float32),
    ),
)
def _edge_geom(posx_hbm, posy_hbm, posz_hbm, ridx_hbm, cidx_hbm,
               dx_hbm, dy_hbm, dz_hbm,
               px, py, pz, rib, cib, dxb, dyb, dzb):
    _, _, base = _worker_base()

    pltpu.sync_copy(posx_hbm, px)
    pltpu.sync_copy(posy_hbm, py)
    pltpu.sync_copy(posz_hbm, pz)

    def geom_group(g):
        rv = rib[pl.ds(g * 16, 16)]
        cv = cib[pl.ds(g * 16, 16)]
        dx = plsc.load_gather(px, [rv]) - plsc.load_gather(px, [cv])
        dy = plsc.load_gather(py, [rv]) - plsc.load_gather(py, [cv])
        dz = plsc.load_gather(pz, [rv]) - plsc.load_gather(pz, [cv])
        dxb[pl.ds(g * 16, 16)] = dx
        dyb[pl.ds(g * 16, 16)] = dy
        dzb[pl.ds(g * 16, 16)] = dz

    @pl.loop(0, NB)
    def _(j):
        off = base + j * K
        pltpu.sync_copy(ridx_hbm.at[pl.ds(off, K)], rib)
        pltpu.sync_copy(cidx_hbm.at[pl.ds(off, K)], cib)
        for g in range(K // 16):
            geom_group(g)
        pltpu.sync_copy(dxb, dx_hbm.at[pl.ds(off, K)])
        pltpu.sync_copy(dyb, dy_hbm.at[pl.ds(off, K)])
        pltpu.sync_copy(dzb, dz_hbm.at[pl.ds(off, K)])

    off = base + NB * K
    pltpu.sync_copy(ridx_hbm.at[pl.ds(off, REM)], rib.at[pl.ds(0, REM)])
    pltpu.sync_copy(cidx_hbm.at[pl.ds(off, REM)], cib.at[pl.ds(0, REM)])
    for g in range(REM // 16):
        geom_group(g)
    pltpu.sync_copy(dxb.at[pl.ds(0, REM)], dx_hbm.at[pl.ds(off, REM)])
    pltpu.sync_copy(dyb.at[pl.ds(0, REM)], dy_hbm.at[pl.ds(off, REM)])
    pltpu.sync_copy(dzb.at[pl.ds(0, REM)], dz_hbm.at[pl.ds(off, REM)])


@functools.partial(
    pl.kernel,
    out_type=jax.ShapeDtypeStruct((NC, NP, D), jnp.float32),
    mesh=_mesh,
    compiler_params=_sc_params,
    scratch_types=(
        pltpu.VMEM_SHARED((NP, D), jnp.float32),
        pltpu.VMEM((K,), jnp.int32),
        pltpu.VMEM((REM,), jnp.int32),
        pltpu.VMEM((K, D), jnp.float32),
        pltpu.VMEM((REM, D), jnp.float32),
    ),
)
def _edge_counts(cidx_hbm, craw_hbm, cacc, ci, cit, ones, onest):
    c, s, base = _worker_base()

    _fill_rows(ones, K, D, 0.0)
    _zero_acc_slice(cacc, ones, s)
    _fill_rows(ones, K, D, 1.0)
    _fill_rows(onest, REM, D, 1.0)
    plsc.subcore_barrier()

    @pl.loop(0, NB)
    def _(j):
        pltpu.sync_copy(cidx_hbm.at[pl.ds(base + j * K, K)], ci)
        pltpu.sync_copy(ones, cacc.at[ci], add=True)

    pltpu.sync_copy(cidx_hbm.at[pl.ds(base + NB * K, REM)], cit)
    pltpu.sync_copy(onest, cacc.at[cit], add=True)

    plsc.subcore_barrier()
    _read_out_acc(cacc, ones, craw_hbm, c, s)


def _make_seg_scatter(linear_src):
    scratch = [
        pltpu.VMEM_SHARED((NP, D), jnp.float32),
        pltpu.VMEM((K,), jnp.int32),
        pltpu.VMEM((REM,), jnp.int32),
        pltpu.VMEM((K, D), jnp.float32),
        pltpu.VMEM((REM, D), jnp.float32),
        pltpu.SemaphoreType.DMA,
    ]
    if not linear_src:
        scratch += [pltpu.VMEM((K,), jnp.int32),
                    pltpu.VMEM((REM,), jnp.int32)]

    def body(*refs):
        it = iter(refs)
        table_hbm = next(it)
        gidx_hbm = None if linear_src else next(it)
        cidx_hbm = next(it)
        out_hbm = next(it)
        acc, ci, cit, rows, rowst, sem = (next(it) for _ in range(6))
        if not linear_src:
            gi, git = next(it), next(it)

        c, s, base = _worker_base()

        _fill_rows(rows, K, D, 0.0)
        _zero_acc_slice(acc, rows, s)
        plsc.subcore_barrier()

        @pl.loop(0, NB)
        def _(j):
            off = base + j * K
            pltpu.sync_copy(cidx_hbm.at[pl.ds(off, K)], ci)
            if linear_src:
                pltpu.sync_copy(table_hbm.at[pl.ds(off, K)], rows)
            else:
                pltpu.sync_copy(gidx_hbm.at[pl.ds(off, K)], gi)
                pltpu.async_copy(table_hbm.at[gi], rows, sem).wait()
            pltpu.sync_copy(rows, acc.at[ci], add=True)

        off = base + NB * K
        pltpu.sync_copy(cidx_hbm.at[pl.ds(off, REM)], cit)
        if linear_src:
            pltpu.sync_copy(table_hbm.at[pl.ds(off, REM)], rowst)
        else:
            pltpu.sync_copy(gidx_hbm.at[pl.ds(off, REM)], git)
            pltpu.async_copy(table_hbm.at[git], rowst, sem).wait()
        pltpu.sync_copy(rowst, acc.at[cit], add=True)

        plsc.subcore_barrier()
        _read_out_acc(acc, rows, out_hbm, c, s)

    return pl.kernel(
        body,
        out_type=jax.ShapeDtypeStruct((NC, NP, D), jnp.float32),
        mesh=_mesh,
        compiler_params=_sc_params,
        scratch_types=tuple(scratch),
    )


_edge_seg_sum = _make_seg_scatter(linear_src=True)
_gather_seg_sum = _make_seg_scatter(linear_src=False)


BE = 4000
BN = 2000


def _x_body(dx_ref, dy_ref, dz_ref, w1_ref, b1_ref, out_ref):
    dx, dy, dz = dx_ref[...], dy_ref[...], dz_ref[...]
    dist = jnp.sqrt(dx * dx + dy * dy + dz * dz + 1e-12)
    w1 = w1_ref[...]
    acc = (b1_ref[...] + dx * w1[0:1, :] + dy * w1[1:2, :]
           + dz * w1[2:3, :] + dist * w1[3:4, :])
    out_ref[...] = jnp.maximum(acc, 0.0)


def _x_tc(dx, dy, dz, W1e, b1e):
    return pl.pallas_call(
        _x_body,
        grid=(E // BE,),
        in_specs=[
            pl.BlockSpec((BE, 1), lambda i: (i, 0)),
            pl.BlockSpec((BE, 1), lambda i: (i, 0)),
            pl.BlockSpec((BE, 1), lambda i: (i, 0)),
            pl.BlockSpec((4, D), lambda i: (0, 0)),
            pl.BlockSpec((1, D), lambda i: (0, 0)),
        ],
        out_specs=pl.BlockSpec((BE, D), lambda i: (i, 0)),
        out_shape=jax.ShapeDtypeStruct((E, D), jnp.float32),
    )(dx, dy, dz, W1e, b1e.reshape(1, D))


def _h0_body(nf_ref, wp_ref, bp_ref, out_ref):
    out_ref[...] = (
        jnp.dot(nf_ref[...], wp_ref[...], preferred_element_type=jnp.float32)
        + bp_ref[...]
    )


def _h0_tc(node_feat, Wp, bp):
    return pl.pallas_call(
        _h0_body,
        grid=(N // BN,),
        in_specs=[
            pl.BlockSpec((BN, D), lambda i: (i, 0)),
            pl.BlockSpec((D, D), lambda i: (0, 0)),
            pl.BlockSpec((1, D), lambda i: (0, 0)),
        ],
        out_specs=pl.BlockSpec((BN, D), lambda i: (i, 0)),
        out_shape=jax.ShapeDtypeStruct((N, D), jnp.float32),
    )(node_feat, Wp, bp.reshape(1, D))


def _s_body(sxp_ref, cp_ref, w2_ref, b2_ref, out_ref):
    sx = sxp_ref[0] + sxp_ref[1]
    craw = cp_ref[0, :, 0:1] + cp_ref[1, :, 0:1]
    out_ref[...] = (
        jnp.dot(sx, w2_ref[...], preferred_element_type=jnp.float32)
        + craw * b2_ref[...]
    )


def _s_tc(SXp, cntp, W2e, b2e):
    return pl.pallas_call(
        _s_body,
        grid=(N // BN,),
        in_specs=[
            pl.BlockSpec((2, BN, D), lambda i: (0, i, 0)),
            pl.BlockSpec((2, BN, D), lambda i: (0, i, 0)),
            pl.BlockSpec((D, D), lambda i: (0, 0)),
            pl.BlockSpec((1, D), lambda i: (0, 0)),
        ],
        out_specs=pl.BlockSpec((BN, D), lambda i: (i, 0)),
        out_shape=jax.ShapeDtypeStruct((N, D), jnp.float32),
    )(SXp, cntp, W2e, b2e.reshape(1, D))


def _layer_body(gp_ref, s_ref, h_ref, cp_ref, wc_ref, bc_ref, out_ref):
    g = gp_ref[0] + gp_ref[1]
    craw = cp_ref[0, :, 0:1] + cp_ref[1, :, 0:1]
    cnt = jnp.maximum(craw, 1.0)
    agg = (
        jnp.dot(g, wc_ref[:D, :], preferred_element_type=jnp.float32)
        + jnp.dot(s_ref[...], wc_ref[D:, :], preferred_element_type=jnp.float32)
        + craw * bc_ref[...]
    ) / cnt
    out_ref[...] = jnp.maximum(h_ref[...] + agg, 0.0)


def _layer_tc(Gp, S, h, cntp, Wc, bc):
    return pl.pallas_call(
        _layer_body,
        grid=(N // BN,),
        in_specs=[
            pl.BlockSpec((2, BN, D), lambda i: (0, i, 0)),
            pl.BlockSpec((BN, D), lambda i: (i, 0)),
            pl.BlockSpec((BN, D), lambda i: (i, 0)),
            pl.BlockSpec((2, BN, D), lambda i: (0, i, 0)),
            pl.BlockSpec((2 * D, D), lambda i: (0, 0)),
            pl.BlockSpec((1, D), lambda i: (0, 0)),
        ],
        out_specs=pl.BlockSpec((BN, D), lambda i: (i, 0)),
        out_shape=jax.ShapeDtypeStruct((N, D), jnp.float32),
    )(Gp, S, h, cntp, Wc, bc.reshape(1, D))


@jax.jit
def kernel(node_feat, pos, edge_index, Wp, bp, W1e, b1e, W2e, b2e,
           Wc0, bc0, Wc1, bc1, Wc2, bc2):
    row = edge_index[0]
    col = edge_index[1]
    posx = pos[:, 0]
    posy = pos[:, 1]
    posz = pos[:, 2]

    dx, dy, dz = _edge_geom(posx, posy, posz, row, col)
    cntp = _edge_counts(col)
    X = _x_tc(dx.reshape(E, 1), dy.reshape(E, 1), dz.reshape(E, 1), W1e, b1e)
    h = _h0_tc(node_feat, Wp, bp)
    SXp = _edge_seg_sum(X, col)
    S = _s_tc(SXp, cntp, W2e, b2e)
    for Wc, bc in ((Wc0, bc0), (Wc1, bc1), (Wc2, bc2)):
        Gp = _gather_seg_sum(h, row, col)
        h = _layer_tc(Gp, S, h, cntp, Wc, bc)
    return h


# speedup vs baseline: 5.3762x; 1.4061x over previous
import functools

import jax
import jax.numpy as jnp
from jax import lax
from jax.experimental import pallas as pl
from jax.experimental.pallas import tpu as pltpu
from jax.experimental.pallas import tpu_sc as plsc

N = 10000
E = 320000
D = 128
NC = 2
NS = 16
NW = NC * NS
K = 128
NBW = 78
XTRA = E // K - NW * NBW
NP = 10240
RPT = NP // NS
ZCH = 128
GS = 6

_mesh = plsc.VectorSubcoreMesh(core_axis_name="c", subcore_axis_name="s")
_sc_params = pltpu.CompilerParams(needs_layout_passes=False)


def _worker_base():
    c = lax.axis_index("c")
    s = lax.axis_index("s")
    wid = c * NS + s
    return c, s, wid, wid * (NBW * K)


def _fill_rows(ref, nrows, ncols, value):
    vec = jnp.full((16,), value, jnp.float32)

    @pl.loop(0, nrows)
    def _(r):
        for cc in range(ncols // 16):
            ref[r, pl.ds(cc * 16, 16)] = vec


def _zero_acc_slice(acc, zbuf, s):
    for k in range(RPT // ZCH):
        pltpu.sync_copy(zbuf, acc.at[pl.ds(s * RPT + k * ZCH, ZCH)])


def _read_out_acc(acc, zbuf, out_hbm, c, s):
    for k in range(RPT // ZCH):
        r0 = s * RPT + k * ZCH
        pltpu.sync_copy(acc.at[pl.ds(r0, ZCH)], zbuf)
        pltpu.sync_copy(zbuf, out_hbm.at[c, pl.ds(r0, ZCH)])


@functools.partial(
    pl.kernel,
    out_type=(
        jax.ShapeDtypeStruct((E,), jnp.float32),
        jax.ShapeDtypeStruct((E,), jnp.float32),
        jax.ShapeDtypeStruct((E,), jnp.float32),
    ),
    mesh=_mesh,
    compiler_params=_sc_params,
    scratch_types=(
        pltpu.VMEM((N,), jnp.float32),
        pltpu.VMEM((N,), jnp.float32),
        pltpu.VMEM((N,), jnp.float32),
        pltpu.VMEM((GS * K,), jnp.int32),
        pltpu.VMEM((GS * K,), jnp.int32),
        pltpu.VMEM((GS * K,), jnp.float32),
        pltpu.VMEM((GS * K,), jnp.float32),
        pltpu.VMEM((GS * K,), jnp.float32),
    ),
)
def _edge_geom(posx_hbm, posy_hbm, posz_hbm, ridx_hbm, cidx_hbm,
               dx_hbm, dy_hbm, dz_hbm,
               px, py, pz, rib, cib, dxb, dyb, dzb):
    _, _, wid, base = _worker_base()

    pltpu.sync_copy(posx_hbm, px)
    pltpu.sync_copy(posy_hbm, py)
    pltpu.sync_copy(posz_hbm, pz)

    def geom_groups(ngroups):
        for g in range(ngroups):
            rv = rib[pl.ds(g * 16, 16)]
            cv = cib[pl.ds(g * 16, 16)]
            dx = plsc.load_gather(px, [rv]) - plsc.load_gather(px, [cv])
            dy = plsc.load_gather(py, [rv]) - plsc.load_gather(py, [cv])
            dz = plsc.load_gather(pz, [rv]) - plsc.load_gather(pz, [cv])
            dxb[pl.ds(g * 16, 16)] = dx
            dyb[pl.ds(g * 16, 16)] = dy
            dzb[pl.ds(g * 16, 16)] = dz

    @pl.loop(0, NBW, step=GS)
    def _(j):
        off = base + j * K
        pltpu.sync_copy(ridx_hbm.at[pl.ds(off, GS * K)], rib)
        pltpu.sync_copy(cidx_hbm.at[pl.ds(off, GS * K)], cib)
        geom_groups(GS * K // 16)
        pltpu.sync_copy(dxb, dx_hbm.at[pl.ds(off, GS * K)])
        pltpu.sync_copy(dyb, dy_hbm.at[pl.ds(off, GS * K)])
        pltpu.sync_copy(dzb, dz_hbm.at[pl.ds(off, GS * K)])

    @pl.when(wid < XTRA)
    def _():
        off = (NW * NBW + wid) * K
        pltpu.sync_copy(ridx_hbm.at[pl.ds(off, K)], rib.at[pl.ds(0, K)])
        pltpu.sync_copy(cidx_hbm.at[pl.ds(off, K)], cib.at[pl.ds(0, K)])
        geom_groups(K // 16)
        pltpu.sync_copy(dxb.at[pl.ds(0, K)], dx_hbm.at[pl.ds(off, K)])
        pltpu.sync_copy(dyb.at[pl.ds(0, K)], dy_hbm.at[pl.ds(off, K)])
        pltpu.sync_copy(dzb.at[pl.ds(0, K)], dz_hbm.at[pl.ds(off, K)])


@functools.partial(
    pl.kernel,
    out_type=jax.ShapeDtypeStruct((NC, NP, D), jnp.float32),
    mesh=_mesh,
    compiler_params=_sc_params,
    scratch_types=(
        pltpu.VMEM_SHARED((NP, D), jnp.float32),
        pltpu.VMEM((K,), jnp.int32),
        pltpu.VMEM((K,), jnp.int32),
        pltpu.VMEM((K, D), jnp.float32),
        pltpu.SemaphoreType.DMA,
        pltpu.SemaphoreType.DMA,
    ),
)
def _edge_counts(cidx_hbm, craw_hbm, cacc, cib0, cib1, ones, ssem0, ssem1):
    c, s, wid, base = _worker_base()

    _fill_rows(ones, K, D, 0.0)
    _zero_acc_slice(cacc, ones, s)
    _fill_rows(ones, K, D, 1.0)
    plsc.subcore_barrier()

    @pl.loop(0, NBW, step=2)
    def _(j):
        off = base + j * K
        pltpu.sync_copy(cidx_hbm.at[pl.ds(off, K)], cib0)
        s0 = pltpu.async_copy(ones, cacc.at[cib0], ssem0, add=True)
        pltpu.sync_copy(cidx_hbm.at[pl.ds(off + K, K)], cib1)
        s1 = pltpu.async_copy(ones, cacc.at[cib1], ssem1, add=True)
        s0.wait()
        s1.wait()

    @pl.when(wid < XTRA)
    def _():
        off = (NW * NBW + wid) * K
        pltpu.sync_copy(cidx_hbm.at[pl.ds(off, K)], cib0)
        pltpu.sync_copy(ones, cacc.at[cib0], add=True)

    plsc.subcore_barrier()
    _read_out_acc(cacc, ones, craw_hbm, c, s)


def _make_seg_scatter(linear_src):
    scratch = [
        pltpu.VMEM_SHARED((NP, D), jnp.float32),
        pltpu.VMEM((K,), jnp.int32),
        pltpu.VMEM((K,), jnp.int32),
        pltpu.VMEM((K, D), jnp.float32),
        pltpu.VMEM((K, D), jnp.float32),
        pltpu.SemaphoreType.DMA,
        pltpu.SemaphoreType.DMA,
        pltpu.SemaphoreType.DMA,
        pltpu.SemaphoreType.DMA,
    ]
    if not linear_src:
        scratch += [pltpu.VMEM((K,), jnp.int32),
                    pltpu.VMEM((K,), jnp.int32)]

    def body(*refs):
        it = iter(refs)
        table_hbm = next(it)
        gidx_hbm = None if linear_src else next(it)
        cidx_hbm = next(it)
        out_hbm = next(it)
        acc, cib0, cib1, rows0, rows1 = (next(it) for _ in range(5))
        gsem0, gsem1, ssem0, ssem1 = (next(it) for _ in range(4))
        if not linear_src:
            gib0, gib1 = next(it), next(it)

        c, s, wid, base = _worker_base()

        _fill_rows(rows0, K, D, 0.0)
        _zero_acc_slice(acc, rows0, s)
        plsc.subcore_barrier()

        def start_gather(off, gib, rows, gsem):
            if linear_src:
                return pltpu.async_copy(table_hbm.at[pl.ds(off, K)], rows,
                                        gsem)
            pltpu.sync_copy(gidx_hbm.at[pl.ds(off, K)], gib)
            return pltpu.async_copy(table_hbm.at[gib], rows, gsem)

        @pl.loop(0, NBW, step=2)
        def _(j):
            off = base + j * K
            pltpu.sync_copy(cidx_hbm.at[pl.ds(off, K)], cib0)
            g0 = start_gather(off, None if linear_src else gib0, rows0, gsem0)
            pltpu.sync_copy(cidx_hbm.at[pl.ds(off + K, K)], cib1)
            g1 = start_gather(off + K, None if linear_src else gib1, rows1,
                              gsem1)
            g0.wait()
            s0 = pltpu.async_copy(rows0, acc.at[cib0], ssem0, add=True)
            g1.wait()
            s1 = pltpu.async_copy(rows1, acc.at[cib1], ssem1, add=True)
            s0.wait()
            s1.wait()

        @pl.when(wid < XTRA)
        def _():
            off = (NW * NBW + wid) * K
            pltpu.sync_copy(cidx_hbm.at[pl.ds(off, K)], cib0)
            g0 = start_gather(off, None if linear_src else gib0, rows0, gsem0)
            g0.wait()
            pltpu.sync_copy(rows0, acc.at[cib0], add=True)

        plsc.subcore_barrier()
        _read_out_acc(acc, rows0, out_hbm, c, s)

    return pl.kernel(
        body,
        out_type=jax.ShapeDtypeStruct((NC, NP, D), jnp.float32),
        mesh=_mesh,
        compiler_params=_sc_params,
        scratch_types=tuple(scratch),
    )


_edge_seg_sum = _make_seg_scatter(linear_src=True)
_gather_seg_sum = _make_seg_scatter(linear_src=False)


BE = 4000
BN = 2000


def _x_body(dx_ref, dy_ref, dz_ref, w1_ref, b1_ref, out_ref):
    dx, dy, dz = dx_ref[...], dy_ref[...], dz_ref[...]
    dist = jnp.sqrt(dx * dx + dy * dy + dz * dz + 1e-12)
    w1 = w1_ref[...]
    acc = (b1_ref[...] + dx * w1[0:1, :] + dy * w1[1:2, :]
           + dz * w1[2:3, :] + dist * w1[3:4, :])
    out_ref[...] = jnp.maximum(acc, 0.0)


def _x_tc(dx, dy, dz, W1e, b1e):
    return pl.pallas_call(
        _x_body,
        grid=(E // BE,),
        in_specs=[
            pl.BlockSpec((BE, 1), lambda i: (i, 0)),
            pl.BlockSpec((BE, 1), lambda i: (i, 0)),
            pl.BlockSpec((BE, 1), lambda i: (i, 0)),
            pl.BlockSpec((4, D), lambda i: (0, 0)),
            pl.BlockSpec((1, D), lambda i: (0, 0)),
        ],
        out_specs=pl.BlockSpec((BE, D), lambda i: (i, 0)),
        out_shape=jax.ShapeDtypeStruct((E, D), jnp.float32),
    )(dx, dy, dz, W1e, b1e.reshape(1, D))


def _h0_body(nf_ref, wp_ref, bp_ref, out_ref):
    out_ref[...] = (
        jnp.dot(nf_ref[...], wp_ref[...], preferred_element_type=jnp.float32)
        + bp_ref[...]
    )


def _h0_tc(node_feat, Wp, bp):
    return pl.pallas_call(
        _h0_body,
        grid=(N // BN,),
        in_specs=[
            pl.BlockSpec((BN, D), lambda i: (i, 0)),
            pl.BlockSpec((D, D), lambda i: (0, 0)),
            pl.BlockSpec((1, D), lambda i: (0, 0)),
        ],
        out_specs=pl.BlockSpec((BN, D), lambda i: (i, 0)),
        out_shape=jax.ShapeDtypeStruct((N, D), jnp.float32),
    )(node_feat, Wp, bp.reshape(1, D))


def _layer_body(gp_ref, sxp_ref, h_ref, cp_ref, w2_ref, b2_ref, wc_ref,
                bc_ref, out_ref):
    g = gp_ref[0] + gp_ref[1]
    sx = sxp_ref[0] + sxp_ref[1]
    craw = cp_ref[0, :, 0:1] + cp_ref[1, :, 0:1]
    cnt = jnp.maximum(craw, 1.0)
    se = (jnp.dot(sx, w2_ref[...], preferred_element_type=jnp.float32)
          + craw * b2_ref[...])
    agg = (
        jnp.dot(g, wc_ref[:D, :], preferred_element_type=jnp.float32)
        + jnp.dot(se, wc_ref[D:, :], preferred_element_type=jnp.float32)
        + craw * bc_ref[...]
    ) / cnt
    out_ref[...] = jnp.maximum(h_ref[...] + agg, 0.0)


def _layer_tc(Gp, SXp, h, cntp, W2e, b2e, Wc, bc):
    return pl.pallas_call(
        _layer_body,
        grid=(N // BN,),
        in_specs=[
            pl.BlockSpec((2, BN, D), lambda i: (0, i, 0)),
            pl.BlockSpec((2, BN, D), lambda i: (0, i, 0)),
            pl.BlockSpec((BN, D), lambda i: (i, 0)),
            pl.BlockSpec((2, BN, D), lambda i: (0, i, 0)),
            pl.BlockSpec((D, D), lambda i: (0, 0)),
            pl.BlockSpec((1, D), lambda i: (0, 0)),
            pl.BlockSpec((2 * D, D), lambda i: (0, 0)),
            pl.BlockSpec((1, D), lambda i: (0, 0)),
        ],
        out_specs=pl.BlockSpec((BN, D), lambda i: (i, 0)),
        out_shape=jax.ShapeDtypeStruct((N, D), jnp.float32),
    )(Gp, SXp, h, cntp, W2e, b2e.reshape(1, D), Wc, bc.reshape(1, D))


@jax.jit
def kernel(node_feat, pos, edge_index, Wp, bp, W1e, b1e, W2e, b2e,
           Wc0, bc0, Wc1, bc1, Wc2, bc2):
    row = edge_index[0]
    col = edge_index[1]
    posx = pos[:, 0]
    posy = pos[:, 1]
    posz = pos[:, 2]

    dx, dy, dz = _edge_geom(posx, posy, posz, row, col)
    cntp = _edge_counts(col)
    X = _x_tc(dx.reshape(E, 1), dy.reshape(E, 1), dz.reshape(E, 1), W1e, b1e)
    h = _h0_tc(node_feat, Wp, bp)
    SXp = _edge_seg_sum(X, col)
    for Wc, bc in ((Wc0, bc0), (Wc1, bc1), (Wc2, bc2)):
        Gp = _gather_seg_sum(h, row, col)
        h = _layer_tc(Gp, SXp, h, cntp, W2e, b2e, Wc, bc)
    return h


# speedup vs baseline: 5.4483x; 1.0134x over previous
import functools

import jax
import jax.numpy as jnp
from jax import lax
from jax.experimental import pallas as pl
from jax.experimental.pallas import tpu as pltpu
from jax.experimental.pallas import tpu_sc as plsc

N = 10000
E = 320000
D = 128
NC = 2
NS = 16
NW = NC * NS
K = 128
NBW = 78
XTRA = E // K - NW * NBW
NP = 10240
RPT = NP // NS
ZCH = 128
GS = NBW

_mesh = plsc.VectorSubcoreMesh(core_axis_name="c", subcore_axis_name="s")
_sc_params = pltpu.CompilerParams(needs_layout_passes=False)


def _worker_base():
    c = lax.axis_index("c")
    s = lax.axis_index("s")
    wid = c * NS + s
    return c, s, wid, wid * (NBW * K)


def _fill_rows(ref, nrows, ncols, value):
    vec = jnp.full((16,), value, jnp.float32)

    @pl.loop(0, nrows)
    def _(r):
        for cc in range(ncols // 16):
            ref[r, pl.ds(cc * 16, 16)] = vec


def _zero_acc_slice(acc, zbuf, s):
    for k in range(RPT // ZCH):
        pltpu.sync_copy(zbuf, acc.at[pl.ds(s * RPT + k * ZCH, ZCH)])


def _read_out_acc(acc, zbuf, out_hbm, c, s):
    for k in range(RPT // ZCH):
        r0 = s * RPT + k * ZCH
        pltpu.sync_copy(acc.at[pl.ds(r0, ZCH)], zbuf)
        pltpu.sync_copy(zbuf, out_hbm.at[c, pl.ds(r0, ZCH)])


@functools.partial(
    pl.kernel,
    out_type=(
        jax.ShapeDtypeStruct((E,), jnp.float32),
        jax.ShapeDtypeStruct((E,), jnp.float32),
        jax.ShapeDtypeStruct((E,), jnp.float32),
    ),
    mesh=_mesh,
    compiler_params=_sc_params,
    scratch_types=(
        pltpu.VMEM((N,), jnp.float32),
        pltpu.VMEM((N,), jnp.float32),
        pltpu.VMEM((N,), jnp.float32),
        pltpu.VMEM((GS * K,), jnp.int32),
        pltpu.VMEM((GS * K,), jnp.int32),
        pltpu.VMEM((GS * K,), jnp.float32),
        pltpu.VMEM((GS * K,), jnp.float32),
        pltpu.VMEM((GS * K,), jnp.float32),
    ),
)
def _edge_geom(posx_hbm, posy_hbm, posz_hbm, ridx_hbm, cidx_hbm,
               dx_hbm, dy_hbm, dz_hbm,
               px, py, pz, rib, cib, dxb, dyb, dzb):
    _, _, wid, base = _worker_base()

    pltpu.sync_copy(posx_hbm, px)
    pltpu.sync_copy(posy_hbm, py)
    pltpu.sync_copy(posz_hbm, pz)

    def geom_groups(ngroups):
        @pl.loop(0, ngroups)
        def _(g):
            rv = rib[pl.ds(g * 16, 16)]
            cv = cib[pl.ds(g * 16, 16)]
            dx = plsc.load_gather(px, [rv]) - plsc.load_gather(px, [cv])
            dy = plsc.load_gather(py, [rv]) - plsc.load_gather(py, [cv])
            dz = plsc.load_gather(pz, [rv]) - plsc.load_gather(pz, [cv])
            dxb[pl.ds(g * 16, 16)] = dx
            dyb[pl.ds(g * 16, 16)] = dy
            dzb[pl.ds(g * 16, 16)] = dz

    off = base
    pltpu.sync_copy(ridx_hbm.at[pl.ds(off, GS * K)], rib)
    pltpu.sync_copy(cidx_hbm.at[pl.ds(off, GS * K)], cib)
    geom_groups(GS * K // 16)
    pltpu.sync_copy(dxb, dx_hbm.at[pl.ds(off, GS * K)])
    pltpu.sync_copy(dyb, dy_hbm.at[pl.ds(off, GS * K)])
    pltpu.sync_copy(dzb, dz_hbm.at[pl.ds(off, GS * K)])

    @pl.when(wid < XTRA)
    def _():
        off2 = (NW * NBW + wid) * K
        pltpu.sync_copy(ridx_hbm.at[pl.ds(off2, K)], rib.at[pl.ds(0, K)])
        pltpu.sync_copy(cidx_hbm.at[pl.ds(off2, K)], cib.at[pl.ds(0, K)])
        geom_groups(K // 16)
        pltpu.sync_copy(dxb.at[pl.ds(0, K)], dx_hbm.at[pl.ds(off2, K)])
        pltpu.sync_copy(dyb.at[pl.ds(0, K)], dy_hbm.at[pl.ds(off2, K)])
        pltpu.sync_copy(dzb.at[pl.ds(0, K)], dz_hbm.at[pl.ds(off2, K)])


@functools.partial(
    pl.kernel,
    out_type=jax.ShapeDtypeStruct((NC, NP, D), jnp.float32),
    mesh=_mesh,
    compiler_params=_sc_params,
    scratch_types=(
        pltpu.VMEM_SHARED((NP, D), jnp.float32),
        pltpu.VMEM((K,), jnp.int32),
        pltpu.VMEM((K,), jnp.int32),
        pltpu.VMEM((K, D), jnp.float32),
        pltpu.SemaphoreType.DMA,
        pltpu.SemaphoreType.DMA,
    ),
)
def _edge_counts(cidx_hbm, craw_hbm, cacc, cib0, cib1, ones, ssem0, ssem1):
    c, s, wid, base = _worker_base()

    _fill_rows(ones, K, D, 0.0)
    _zero_acc_slice(cacc, ones, s)
    _fill_rows(ones, K, D, 1.0)
    plsc.subcore_barrier()

    @pl.loop(0, NBW, step=2)
    def _(j):
        off = base + j * K
        pltpu.sync_copy(cidx_hbm.at[pl.ds(off, K)], cib0)

        @pl.when(j > 0)
        def _():
            pltpu.make_async_copy(craw_hbm.at[0, pl.ds(0, K)], ones,
                                  ssem0).wait()
            pltpu.make_async_copy(craw_hbm.at[0, pl.ds(0, K)], ones,
                                  ssem1).wait()

        pltpu.async_copy(ones, cacc.at[cib0], ssem0, add=True)
        pltpu.sync_copy(cidx_hbm.at[pl.ds(off + K, K)], cib1)
        pltpu.async_copy(ones, cacc.at[cib1], ssem1, add=True)

    pltpu.make_async_copy(craw_hbm.at[0, pl.ds(0, K)], ones, ssem0).wait()
    pltpu.make_async_copy(craw_hbm.at[0, pl.ds(0, K)], ones, ssem1).wait()

    @pl.when(wid < XTRA)
    def _():
        off = (NW * NBW + wid) * K
        pltpu.sync_copy(cidx_hbm.at[pl.ds(off, K)], cib0)
        pltpu.sync_copy(ones, cacc.at[cib0], add=True)

    plsc.subcore_barrier()
    _read_out_acc(cacc, ones, craw_hbm, c, s)


def _make_seg_scatter(linear_src):
    scratch = [
        pltpu.VMEM_SHARED((NP, D), jnp.float32),
        pltpu.VMEM((K,), jnp.int32),
        pltpu.VMEM((K,), jnp.int32),
        pltpu.VMEM((K, D), jnp.float32),
        pltpu.VMEM((K, D), jnp.float32),
        pltpu.SemaphoreType.DMA,
        pltpu.SemaphoreType.DMA,
        pltpu.SemaphoreType.DMA,
        pltpu.SemaphoreType.DMA,
    ]
    if not linear_src:
        scratch += [pltpu.VMEM((K,), jnp.int32),
                    pltpu.VMEM((K,), jnp.int32)]

    def body(*refs):
        it = iter(refs)
        table_hbm = next(it)
        gidx_hbm = None if linear_src else next(it)
        cidx_hbm = next(it)
        out_hbm = next(it)
        acc, cib0, cib1, rows0, rows1 = (next(it) for _ in range(5))
        gsem0, gsem1, ssem0, ssem1 = (next(it) for _ in range(4))
        if not linear_src:
            gib0, gib1 = next(it), next(it)

        c, s, wid, base = _worker_base()

        _fill_rows(rows0, K, D, 0.0)
        _zero_acc_slice(acc, rows0, s)
        plsc.subcore_barrier()

        def start_gather(off, gib, rows, gsem):
            if linear_src:
                return pltpu.async_copy(table_hbm.at[pl.ds(off, K)], rows,
                                        gsem)
            pltpu.sync_copy(gidx_hbm.at[pl.ds(off, K)], gib)
            return pltpu.async_copy(table_hbm.at[gib], rows, gsem)

        @pl.loop(0, NBW, step=2)
        def _(j):
            off = base + j * K

            @pl.when(j > 0)
            def _():
                pltpu.make_async_copy(out_hbm.at[0, pl.ds(0, K)], rows0,
                                      ssem0).wait()
                pltpu.make_async_copy(out_hbm.at[0, pl.ds(0, K)], rows1,
                                      ssem1).wait()

            pltpu.sync_copy(cidx_hbm.at[pl.ds(off, K)], cib0)
            g0 = start_gather(off, None if linear_src else gib0, rows0, gsem0)
            pltpu.sync_copy(cidx_hbm.at[pl.ds(off + K, K)], cib1)
            g1 = start_gather(off + K, None if linear_src else gib1, rows1,
                              gsem1)
            g0.wait()
            pltpu.async_copy(rows0, acc.at[cib0], ssem0, add=True)
            g1.wait()
            pltpu.async_copy(rows1, acc.at[cib1], ssem1, add=True)

        pltpu.make_async_copy(out_hbm.at[0, pl.ds(0, K)], rows0, ssem0).wait()
        pltpu.make_async_copy(out_hbm.at[0, pl.ds(0, K)], rows1, ssem1).wait()

        @pl.when(wid < XTRA)
        def _():
            off = (NW * NBW + wid) * K
            pltpu.sync_copy(cidx_hbm.at[pl.ds(off, K)], cib0)
            g0 = start_gather(off, None if linear_src else gib0, rows0, gsem0)
            g0.wait()
            pltpu.sync_copy(rows0, acc.at[cib0], add=True)

        plsc.subcore_barrier()
        _read_out_acc(acc, rows0, out_hbm, c, s)

    return pl.kernel(
        body,
        out_type=jax.ShapeDtypeStruct((NC, NP, D), jnp.float32),
        mesh=_mesh,
        compiler_params=_sc_params,
        scratch_types=tuple(scratch),
    )


_edge_seg_sum = _make_seg_scatter(linear_src=True)
_gather_seg_sum = _make_seg_scatter(linear_src=False)


BE = 4000
BN = 2000


def _x_body(dx_ref, dy_ref, dz_ref, w1_ref, b1_ref, out_ref):
    dx, dy, dz = dx_ref[...], dy_ref[...], dz_ref[...]
    dist = jnp.sqrt(dx * dx + dy * dy + dz * dz + 1e-12)
    w1 = w1_ref[...]
    acc = (b1_ref[...] + dx * w1[0:1, :] + dy * w1[1:2, :]
           + dz * w1[2:3, :] + dist * w1[3:4, :])
    out_ref[...] = jnp.maximum(acc, 0.0)


def _x_tc(dx, dy, dz, W1e, b1e):
    return pl.pallas_call(
        _x_body,
        grid=(E // BE,),
        in_specs=[
            pl.BlockSpec((BE, 1), lambda i: (i, 0)),
            pl.BlockSpec((BE, 1), lambda i: (i, 0)),
            pl.BlockSpec((BE, 1), lambda i: (i, 0)),
            pl.BlockSpec((4, D), lambda i: (0, 0)),
            pl.BlockSpec((1, D), lambda i: (0, 0)),
        ],
        out_specs=pl.BlockSpec((BE, D), lambda i: (i, 0)),
        out_shape=jax.ShapeDtypeStruct((E, D), jnp.float32),
    )(dx, dy, dz, W1e, b1e.reshape(1, D))


def _h0_body(nf_ref, wp_ref, bp_ref, out_ref):
    out_ref[...] = (
        jnp.dot(nf_ref[...], wp_ref[...], preferred_element_type=jnp.float32)
        + bp_ref[...]
    )


def _h0_tc(node_feat, Wp, bp):
    return pl.pallas_call(
        _h0_body,
        grid=(N // BN,),
        in_specs=[
            pl.BlockSpec((BN, D), lambda i: (i, 0)),
            pl.BlockSpec((D, D), lambda i: (0, 0)),
            pl.BlockSpec((1, D), lambda i: (0, 0)),
        ],
        out_specs=pl.BlockSpec((BN, D), lambda i: (i, 0)),
        out_shape=jax.ShapeDtypeStruct((N, D), jnp.float32),
    )(node_feat, Wp, bp.reshape(1, D))


def _layer_body(gp_ref, sxp_ref, h_ref, cp_ref, w2_ref, b2_ref, wc_ref,
                bc_ref, out_ref):
    g = gp_ref[0] + gp_ref[1]
    sx = sxp_ref[0] + sxp_ref[1]
    craw = cp_ref[0, :, 0:1] + cp_ref[1, :, 0:1]
    cnt = jnp.maximum(craw, 1.0)
    se = (jnp.dot(sx, w2_ref[...], preferred_element_type=jnp.float32)
          + craw * b2_ref[...])
    agg = (
        jnp.dot(g, wc_ref[:D, :], preferred_element_type=jnp.float32)
        + jnp.dot(se, wc_ref[D:, :], preferred_element_type=jnp.float32)
        + craw * bc_ref[...]
    ) / cnt
    out_ref[...] = jnp.maximum(h_ref[...] + agg, 0.0)


def _layer_tc(Gp, SXp, h, cntp, W2e, b2e, Wc, bc):
    return pl.pallas_call(
        _layer_body,
        grid=(N // BN,),
        in_specs=[
            pl.BlockSpec((2, BN, D), lambda i: (0, i, 0)),
            pl.BlockSpec((2, BN, D), lambda i: (0, i, 0)),
            pl.BlockSpec((BN, D), lambda i: (i, 0)),
            pl.BlockSpec((2, BN, D), lambda i: (0, i, 0)),
            pl.BlockSpec((D, D), lambda i: (0, 0)),
            pl.BlockSpec((1, D), lambda i: (0, 0)),
            pl.BlockSpec((2 * D, D), lambda i: (0, 0)),
            pl.BlockSpec((1, D), lambda i: (0, 0)),
        ],
        out_specs=pl.BlockSpec((BN, D), lambda i: (i, 0)),
        out_shape=jax.ShapeDtypeStruct((N, D), jnp.float32),
    )(Gp, SXp, h, cntp, W2e, b2e.reshape(1, D), Wc, bc.reshape(1, D))


@jax.jit
def kernel(node_feat, pos, edge_index, Wp, bp, W1e, b1e, W2e, b2e,
           Wc0, bc0, Wc1, bc1, Wc2, bc2):
    row = edge_index[0]
    col = edge_index[1]
    posx = pos[:, 0]
    posy = pos[:, 1]
    posz = pos[:, 2]

    dx, dy, dz = _edge_geom(posx, posy, posz, row, col)
    cntp = _edge_counts(col)
    X = _x_tc(dx.reshape(E, 1), dy.reshape(E, 1), dz.reshape(E, 1), W1e, b1e)
    h = _h0_tc(node_feat, Wp, bp)
    SXp = _edge_seg_sum(X, col)
    for Wc, bc in ((Wc0, bc0), (Wc1, bc1), (Wc2, bc2)):
        Gp = _gather_seg_sum(h, row, col)
        h = _layer_tc(Gp, SXp, h, cntp, W2e, b2e, Wc, bc)
    return h


# speedup vs baseline: 6.1984x; 1.1377x over previous
import functools

import jax
import jax.numpy as jnp
from jax import lax
from jax.experimental import pallas as pl
from jax.experimental.pallas import tpu as pltpu
from jax.experimental.pallas import tpu_sc as plsc

N = 10000
E = 320000
D = 128
NC = 2
NS = 16
NW = NC * NS
K = 128
NBW = 78
XTRA = E // K - NW * NBW
NP = 10112
RPT = NP // NS
ACHUNKS = ((0, 128), (128, 128), (256, 128), (384, 128), (512, 120))
NBUF = 3
GS = NBW

_mesh = plsc.VectorSubcoreMesh(core_axis_name="c", subcore_axis_name="s")
_sc_params = pltpu.CompilerParams(needs_layout_passes=False)


def _worker_base():
    c = lax.axis_index("c")
    s = lax.axis_index("s")
    wid = c * NS + s
    return c, s, wid, wid * (NBW * K)


def _fill_rows(ref, nrows, ncols, value):
    vec = jnp.full((16,), value, jnp.float32)

    @pl.loop(0, nrows)
    def _(r):
        for cc in range(ncols // 16):
            ref[r, pl.ds(cc * 16, 16)] = vec


def _zero_acc_slice(acc, zbuf, s):
    for r, n in ACHUNKS:
        pltpu.sync_copy(zbuf.at[pl.ds(0, n)], acc.at[pl.ds(s * RPT + r, n)])


def _read_out_acc(acc, zbuf, out_hbm, c, s):
    for r, n in ACHUNKS:
        r0 = s * RPT + r
        pltpu.sync_copy(acc.at[pl.ds(r0, n)], zbuf.at[pl.ds(0, n)])
        pltpu.sync_copy(zbuf.at[pl.ds(0, n)], out_hbm.at[c, pl.ds(r0, n)])


@functools.partial(
    pl.kernel,
    out_type=(
        jax.ShapeDtypeStruct((E,), jnp.float32),
        jax.ShapeDtypeStruct((E,), jnp.float32),
        jax.ShapeDtypeStruct((E,), jnp.float32),
    ),
    mesh=_mesh,
    compiler_params=_sc_params,
    scratch_types=(
        pltpu.VMEM((N,), jnp.float32),
        pltpu.VMEM((N,), jnp.float32),
        pltpu.VMEM((N,), jnp.float32),
        pltpu.VMEM((GS * K,), jnp.int32),
        pltpu.VMEM((GS * K,), jnp.int32),
        pltpu.VMEM((GS * K,), jnp.float32),
        pltpu.VMEM((GS * K,), jnp.float32),
        pltpu.VMEM((GS * K,), jnp.float32),
    ),
)
def _edge_geom(posx_hbm, posy_hbm, posz_hbm, ridx_hbm, cidx_hbm,
               dx_hbm, dy_hbm, dz_hbm,
               px, py, pz, rib, cib, dxb, dyb, dzb):
    _, _, wid, base = _worker_base()

    pltpu.sync_copy(posx_hbm, px)
    pltpu.sync_copy(posy_hbm, py)
    pltpu.sync_copy(posz_hbm, pz)

    def geom_groups(ngroups):
        @pl.loop(0, ngroups)
        def _(g):
            rv = rib[pl.ds(g * 16, 16)]
            cv = cib[pl.ds(g * 16, 16)]
            dx = plsc.load_gather(px, [rv]) - plsc.load_gather(px, [cv])
            dy = plsc.load_gather(py, [rv]) - plsc.load_gather(py, [cv])
            dz = plsc.load_gather(pz, [rv]) - plsc.load_gather(pz, [cv])
            dxb[pl.ds(g * 16, 16)] = dx
            dyb[pl.ds(g * 16, 16)] = dy
            dzb[pl.ds(g * 16, 16)] = dz

    off = base
    pltpu.sync_copy(ridx_hbm.at[pl.ds(off, GS * K)], rib)
    pltpu.sync_copy(cidx_hbm.at[pl.ds(off, GS * K)], cib)
    geom_groups(GS * K // 16)
    pltpu.sync_copy(dxb, dx_hbm.at[pl.ds(off, GS * K)])
    pltpu.sync_copy(dyb, dy_hbm.at[pl.ds(off, GS * K)])
    pltpu.sync_copy(dzb, dz_hbm.at[pl.ds(off, GS * K)])

    @pl.when(wid < XTRA)
    def _():
        off2 = (NW * NBW + wid) * K
        pltpu.sync_copy(ridx_hbm.at[pl.ds(off2, K)], rib.at[pl.ds(0, K)])
        pltpu.sync_copy(cidx_hbm.at[pl.ds(off2, K)], cib.at[pl.ds(0, K)])
        geom_groups(K // 16)
        pltpu.sync_copy(dxb.at[pl.ds(0, K)], dx_hbm.at[pl.ds(off2, K)])
        pltpu.sync_copy(dyb.at[pl.ds(0, K)], dy_hbm.at[pl.ds(off2, K)])
        pltpu.sync_copy(dzb.at[pl.ds(0, K)], dz_hbm.at[pl.ds(off2, K)])


@functools.partial(
    pl.kernel,
    out_type=jax.ShapeDtypeStruct((NC, NP, D), jnp.float32),
    mesh=_mesh,
    compiler_params=_sc_params,
    scratch_types=(
        pltpu.VMEM_SHARED((NP, D), jnp.float32),
        pltpu.VMEM((K,), jnp.int32),
        pltpu.VMEM((K,), jnp.int32),
        pltpu.VMEM((K, D), jnp.float32),
        pltpu.SemaphoreType.DMA,
        pltpu.SemaphoreType.DMA,
    ),
)
def _edge_counts(cidx_hbm, craw_hbm, cacc, cib0, cib1, ones, ssem0, ssem1):
    c, s, wid, base = _worker_base()

    _fill_rows(ones, K, D, 0.0)
    _zero_acc_slice(cacc, ones, s)
    _fill_rows(ones, K, D, 1.0)
    plsc.subcore_barrier()

    @pl.loop(0, NBW, step=2)
    def _(j):
        off = base + j * K
        pltpu.sync_copy(cidx_hbm.at[pl.ds(off, K)], cib0)

        @pl.when(j > 0)
        def _():
            pltpu.make_async_copy(craw_hbm.at[0, pl.ds(0, K)], ones,
                                  ssem0).wait()
            pltpu.make_async_copy(craw_hbm.at[0, pl.ds(0, K)], ones,
                                  ssem1).wait()

        pltpu.async_copy(ones, cacc.at[cib0], ssem0, add=True)
        pltpu.sync_copy(cidx_hbm.at[pl.ds(off + K, K)], cib1)
        pltpu.async_copy(ones, cacc.at[cib1], ssem1, add=True)

    pltpu.make_async_copy(craw_hbm.at[0, pl.ds(0, K)], ones, ssem0).wait()
    pltpu.make_async_copy(craw_hbm.at[0, pl.ds(0, K)], ones, ssem1).wait()

    @pl.when(wid < XTRA)
    def _():
        off = (NW * NBW + wid) * K
        pltpu.sync_copy(cidx_hbm.at[pl.ds(off, K)], cib0)
        pltpu.sync_copy(ones, cacc.at[cib0], add=True)

    plsc.subcore_barrier()
    _read_out_acc(cacc, ones, craw_hbm, c, s)


def _make_seg_scatter(linear_src):
    scratch = [pltpu.VMEM_SHARED((NP, D), jnp.float32)]
    scratch += [pltpu.VMEM((K,), jnp.int32) for _ in range(NBUF)]
    scratch += [pltpu.VMEM((K, D), jnp.float32) for _ in range(NBUF)]
    scratch += [pltpu.SemaphoreType.DMA for _ in range(NBUF)]
    scratch += [pltpu.SemaphoreType.DMA for _ in range(NBUF)]
    if not linear_src:
        scratch += [pltpu.VMEM((K,), jnp.int32) for _ in range(NBUF)]

    def body(*refs):
        it = iter(refs)
        table_hbm = next(it)
        gidx_hbm = None if linear_src else next(it)
        cidx_hbm = next(it)
        out_hbm = next(it)
        acc = next(it)
        cib = [next(it) for _ in range(NBUF)]
        rows = [next(it) for _ in range(NBUF)]
        gsem = [next(it) for _ in range(NBUF)]
        ssem = [next(it) for _ in range(NBUF)]
        gib = None if linear_src else [next(it) for _ in range(NBUF)]

        c, s, wid, base = _worker_base()

        _fill_rows(rows[0], K, D, 0.0)
        _zero_acc_slice(acc, rows[0], s)
        plsc.subcore_barrier()

        def start_gather(off, p):
            if linear_src:
                return pltpu.async_copy(table_hbm.at[pl.ds(off, K)], rows[p],
                                        gsem[p])
            pltpu.sync_copy(gidx_hbm.at[pl.ds(off, K)], gib[p])
            return pltpu.async_copy(table_hbm.at[gib[p]], rows[p], gsem[p])

        def drain(p):
            pltpu.make_async_copy(out_hbm.at[0, pl.ds(0, K)], rows[p],
                                  ssem[p]).wait()

        gd = {}
        for b in range(NBW):
            p = b % NBUF
            if b >= NBUF:
                drain(p)
            pltpu.sync_copy(cidx_hbm.at[pl.ds(base + b * K, K)], cib[p])
            gd[b] = start_gather(base + b * K, p)
            if b >= 1:
                q = (b - 1) % NBUF
                gd[b - 1].wait()
                pltpu.async_copy(rows[q], acc.at[cib[q]], ssem[q], add=True)
        q = (NBW - 1) % NBUF
        gd[NBW - 1].wait()
        pltpu.async_copy(rows[q], acc.at[cib[q]], ssem[q], add=True)
        for b in range(NBW - NBUF, NBW):
            drain(b % NBUF)

        @pl.when(wid < XTRA)
        def _():
            off = (NW * NBW + wid) * K
            pltpu.sync_copy(cidx_hbm.at[pl.ds(off, K)], cib[0])
            g0 = start_gather(off, 0)
            g0.wait()
            pltpu.sync_copy(rows[0], acc.at[cib[0]], add=True)

        plsc.subcore_barrier()
        _read_out_acc(acc, rows[0], out_hbm, c, s)

    return pl.kernel(
        body,
        out_type=jax.ShapeDtypeStruct((NC, NP, D), jnp.float32),
        mesh=_mesh,
        compiler_params=_sc_params,
        scratch_types=tuple(scratch),
    )


_edge_seg_sum = _make_seg_scatter(linear_src=True)
_gather_seg_sum = _make_seg_scatter(linear_src=False)


BE = 4000
BN = 2000


def _x_body(dx_ref, dy_ref, dz_ref, w1_ref, b1_ref, out_ref):
    dx, dy, dz = dx_ref[...], dy_ref[...], dz_ref[...]
    dist = jnp.sqrt(dx * dx + dy * dy + dz * dz + 1e-12)
    w1 = w1_ref[...]
    acc = (b1_ref[...] + dx * w1[0:1, :] + dy * w1[1:2, :]
           + dz * w1[2:3, :] + dist * w1[3:4, :])
    out_ref[...] = jnp.maximum(acc, 0.0)


def _x_tc(dx, dy, dz, W1e, b1e):
    return pl.pallas_call(
        _x_body,
        grid=(E // BE,),
        in_specs=[
            pl.BlockSpec((BE, 1), lambda i: (i, 0)),
            pl.BlockSpec((BE, 1), lambda i: (i, 0)),
            pl.BlockSpec((BE, 1), lambda i: (i, 0)),
            pl.BlockSpec((4, D), lambda i: (0, 0)),
            pl.BlockSpec((1, D), lambda i: (0, 0)),
        ],
        out_specs=pl.BlockSpec((BE, D), lambda i: (i, 0)),
        out_shape=jax.ShapeDtypeStruct((E, D), jnp.float32),
    )(dx, dy, dz, W1e, b1e.reshape(1, D))


def _h0_body(nf_ref, wp_ref, bp_ref, out_ref):
    out_ref[...] = (
        jnp.dot(nf_ref[...], wp_ref[...], preferred_element_type=jnp.float32)
        + bp_ref[...]
    )


def _h0_tc(node_feat, Wp, bp):
    return pl.pallas_call(
        _h0_body,
        grid=(N // BN,),
        in_specs=[
            pl.BlockSpec((BN, D), lambda i: (i, 0)),
            pl.BlockSpec((D, D), lambda i: (0, 0)),
            pl.BlockSpec((1, D), lambda i: (0, 0)),
        ],
        out_specs=pl.BlockSpec((BN, D), lambda i: (i, 0)),
        out_shape=jax.ShapeDtypeStruct((N, D), jnp.float32),
    )(node_feat, Wp, bp.reshape(1, D))


def _layer_body(gp_ref, sxp_ref, h_ref, cp_ref, w2_ref, b2_ref, wc_ref,
                bc_ref, out_ref):
    g = gp_ref[0] + gp_ref[1]
    sx = sxp_ref[0] + sxp_ref[1]
    craw = cp_ref[0, :, 0:1] + cp_ref[1, :, 0:1]
    cnt = jnp.maximum(craw, 1.0)
    se = (jnp.dot(sx, w2_ref[...], preferred_element_type=jnp.float32)
          + craw * b2_ref[...])
    agg = (
        jnp.dot(g, wc_ref[:D, :], preferred_element_type=jnp.float32)
        + jnp.dot(se, wc_ref[D:, :], preferred_element_type=jnp.float32)
        + craw * bc_ref[...]
    ) / cnt
    out_ref[...] = jnp.maximum(h_ref[...] + agg, 0.0)


def _layer_tc(Gp, SXp, h, cntp, W2e, b2e, Wc, bc):
    return pl.pallas_call(
        _layer_body,
        grid=(N // BN,),
        in_specs=[
            pl.BlockSpec((2, BN, D), lambda i: (0, i, 0)),
            pl.BlockSpec((2, BN, D), lambda i: (0, i, 0)),
            pl.BlockSpec((BN, D), lambda i: (i, 0)),
            pl.BlockSpec((2, BN, D), lambda i: (0, i, 0)),
            pl.BlockSpec((D, D), lambda i: (0, 0)),
            pl.BlockSpec((1, D), lambda i: (0, 0)),
            pl.BlockSpec((2 * D, D), lambda i: (0, 0)),
            pl.BlockSpec((1, D), lambda i: (0, 0)),
        ],
        out_specs=pl.BlockSpec((BN, D), lambda i: (i, 0)),
        out_shape=jax.ShapeDtypeStruct((N, D), jnp.float32),
    )(Gp, SXp, h, cntp, W2e, b2e.reshape(1, D), Wc, bc.reshape(1, D))


@jax.jit
def kernel(node_feat, pos, edge_index, Wp, bp, W1e, b1e, W2e, b2e,
           Wc0, bc0, Wc1, bc1, Wc2, bc2):
    row = edge_index[0]
    col = edge_index[1]
    posx = pos[:, 0]
    posy = pos[:, 1]
    posz = pos[:, 2]

    dx, dy, dz = _edge_geom(posx, posy, posz, row, col)
    cntp = _edge_counts(col)
    X = _x_tc(dx.reshape(E, 1), dy.reshape(E, 1), dz.reshape(E, 1), W1e, b1e)
    h = _h0_tc(node_feat, Wp, bp)
    SXp = _edge_seg_sum(X, col)
    for Wc, bc in ((Wc0, bc0), (Wc1, bc1), (Wc2, bc2)):
        Gp = _gather_seg_sum(h, row, col)
        h = _layer_tc(Gp, SXp, h, cntp, W2e, b2e, Wc, bc)
    return h
